# Initial kernel scaffold; baseline (speedup 1.0000x reference)
import jax
import jax.numpy as jnp
from jax.experimental import pallas as pl


def kernel(x, edge_index, Wl1, bl1, Wr1, Wl2, bl2, Wr2, Wl3, bl3, Wr3, Wl4, bl4, Wr4, Wl5, bl5, Wr5):
    raise NotImplementedError("write your pallas kernel here")


# speedup vs baseline: 4.3543x; 4.3543x over previous
import functools

import jax
import jax.numpy as jnp
from jax import lax
from jax.experimental import pallas as pl
from jax.experimental.pallas import tpu as pltpu
from jax.experimental.pallas import tpu_sc as plsc

N = 10000
E = 320000
D = 128

NC = 2
NS = 16
CHUNK = 80
EDGES_PER_TILE = E // (NC * NS)
NCHUNKS = EDGES_PER_TILE // CHUNK
RPT = 624


def _build_seg_sum(ones_mode: bool):
    mesh = plsc.VectorSubcoreMesh(
        core_axis_name="c", subcore_axis_name="s", num_cores=NC, num_subcores=NS
    )

    @functools.partial(
        pl.kernel,
        out_type=jax.ShapeDtypeStruct((NC, N, D), jnp.float32),
        mesh=mesh,
        scratch_types=[
            pltpu.VMEM((CHUNK,), jnp.int32),
            pltpu.VMEM((CHUNK,), jnp.int32),
            pltpu.VMEM((CHUNK, D), jnp.float32),
            pltpu.VMEM((16, D), jnp.float32),
            pltpu.SemaphoreType.DMA,
            pltpu.VMEM_SHARED((N, D), jnp.float32),
        ],
    )
    def seg_sum(*args):
        if ones_mode:
            (dst_hbm, out_hbm, src_v, dst_v, rows_v, zbuf, sem, agg_sh) = args
        else:
            (table_hbm, src_hbm, dst_hbm, out_hbm, src_v, dst_v, rows_v,
             zbuf, sem, agg_sh) = args
        c = lax.axis_index("c")
        s = lax.axis_index("s")

        zvec = jnp.zeros((16,), jnp.float32)
        ovec = jnp.ones((16,), jnp.float32)

        def zero_fill(i, carry):
            for j in range(D // 16):
                zbuf[i, pl.ds(j * 16, 16)] = zvec
            return carry

        lax.fori_loop(0, 16, zero_fill, None)

        if ones_mode:
            def ones_fill(i, carry):
                for j in range(D // 16):
                    rows_v[i, pl.ds(j * 16, 16)] = ovec
                return carry

            lax.fori_loop(0, CHUNK, ones_fill, None)

        def zero_copy(i, carry):
            pltpu.sync_copy(zbuf, agg_sh.at[pl.ds(s * RPT + i * 16, 16)])
            return carry

        lax.fori_loop(0, RPT // 16, zero_copy, None)

        @pl.when(s == NS - 1)
        def _():
            pltpu.sync_copy(zbuf, agg_sh.at[pl.ds(N - 16, 16)])

        plsc.subcore_barrier()

        ebase = (c * NS + s) * EDGES_PER_TILE

        def body(g, carry):
            off = ebase + g * CHUNK
            if not ones_mode:
                pltpu.sync_copy(src_hbm.at[pl.ds(off, CHUNK)], src_v)
                pltpu.async_copy(table_hbm.at[src_v], rows_v, sem).wait()
            pltpu.sync_copy(dst_hbm.at[pl.ds(off, CHUNK)], dst_v)
            pltpu.sync_copy(rows_v, agg_sh.at[dst_v], add=True)
            return carry

        lax.fori_loop(0, NCHUNKS, body, None)
        plsc.subcore_barrier()

        pltpu.sync_copy(
            agg_sh.at[pl.ds(s * RPT, RPT)],
            out_hbm.at[c].at[pl.ds(s * RPT, RPT)],
        )

        @pl.when(s == NS - 1)
        def _():
            pltpu.sync_copy(
                agg_sh.at[pl.ds(N - 16, 16)],
                out_hbm.at[c].at[pl.ds(N - 16, 16)],
            )

    return seg_sum


_seg_sum_feat = _build_seg_sum(False)
_seg_sum_deg = _build_seg_sum(True)


def _dense(p, h, d0, d1, Wl, bl, Wr, relu):
    BLK = 1000

    def body(p_ref, h_ref, d0_ref, d1_ref, wl_ref, bl_ref, wr_ref, o_ref):
        deg = jnp.maximum(d0_ref[...] + d1_ref[...], 1.0)
        agg = (p_ref[0] + p_ref[1]) / deg
        acc = lax.dot_general(
            agg, wl_ref[...], (((1,), (1,)), ((), ())),
            preferred_element_type=jnp.float32,
        )
        acc = acc + bl_ref[...] + lax.dot_general(
            h_ref[...], wr_ref[...], (((1,), (1,)), ((), ())),
            preferred_element_type=jnp.float32,
        )
        if relu:
            acc = jnp.maximum(acc, 0.0)
        o_ref[...] = acc

    return pl.pallas_call(
        body,
        grid=(N // BLK,),
        in_specs=[
            pl.BlockSpec((NC, BLK, D), lambda i: (0, i, 0)),
            pl.BlockSpec((BLK, D), lambda i: (i, 0)),
            pl.BlockSpec((BLK, 1), lambda i: (i, 0)),
            pl.BlockSpec((BLK, 1), lambda i: (i, 0)),
            pl.BlockSpec((D, D), lambda i: (0, 0)),
            pl.BlockSpec((1, D), lambda i: (0, 0)),
            pl.BlockSpec((D, D), lambda i: (0, 0)),
        ],
        out_specs=pl.BlockSpec((BLK, D), lambda i: (i, 0)),
        out_shape=jax.ShapeDtypeStruct((N, D), jnp.float32),
    )(p, h, d0, d1, Wl, bl, Wr)


def kernel(x, edge_index, Wl1, bl1, Wr1, Wl2, bl2, Wr2, Wl3, bl3, Wr3,
           Wl4, bl4, Wr4, Wl5, bl5, Wr5):
    src = edge_index[0]
    dst = edge_index[1]

    degp = _seg_sum_deg(dst)
    d0 = degp[0, :, 0:1]
    d1 = degp[1, :, 0:1]

    layers = [
        (Wl1, bl1, Wr1, True),
        (Wl2, bl2, Wr2, True),
        (Wl3, bl3, Wr3, True),
        (Wl4, bl4, Wr4, True),
        (Wl5, bl5, Wr5, False),
    ]
    h = x
    for Wl, bl, Wr, relu in layers:
        p = _seg_sum_feat(h, src, dst)
        h = _dense(p, h, d0, d1, Wl, bl.reshape(1, D), Wr, relu)
    return h


# speedup vs baseline: 8.1355x; 1.8684x over previous
import functools

import jax
import jax.numpy as jnp
from jax import lax
from jax.experimental import pallas as pl
from jax.experimental.pallas import tpu as pltpu
from jax.experimental.pallas import tpu_sc as plsc

N = 10000
E = 320000
D = 128

NC = 2
NS = 16
CHUNK = 80
EDGES_PER_TILE = E // (NC * NS)
NCHUNKS = EDGES_PER_TILE // CHUNK
RPT = 624


def _build_seg_sum(ones_mode: bool):
    mesh = plsc.VectorSubcoreMesh(
        core_axis_name="c", subcore_axis_name="s", num_cores=NC, num_subcores=NS
    )

    @functools.partial(
        pl.kernel,
        out_type=jax.ShapeDtypeStruct((NC, N, D), jnp.float32),
        mesh=mesh,
        scratch_types=[
            pltpu.VMEM((2, CHUNK), jnp.int32),
            pltpu.VMEM((2, CHUNK), jnp.int32),
            pltpu.VMEM((2, CHUNK, D), jnp.float32),
            pltpu.VMEM((16, D), jnp.float32),
            pltpu.SemaphoreType.DMA,
            pltpu.SemaphoreType.DMA,
            pltpu.VMEM_SHARED((N, D), jnp.float32),
        ],
    )
    def seg_sum(*args):
        if ones_mode:
            (dst_hbm, out_hbm, src_v, dst_v, rows_v, zbuf, gsem, isem,
             agg_sh) = args
        else:
            (table_hbm, src_hbm, dst_hbm, out_hbm, src_v, dst_v, rows_v,
             zbuf, gsem, isem, agg_sh) = args
        c = lax.axis_index("c")
        s = lax.axis_index("s")

        zvec = jnp.zeros((16,), jnp.float32)
        ovec = jnp.ones((16,), jnp.float32)

        def zero_fill(i, carry):
            for j in range(D // 16):
                zbuf[i, pl.ds(j * 16, 16)] = zvec
            return carry

        lax.fori_loop(0, 16, zero_fill, None)

        if ones_mode:
            def ones_fill(i, carry):
                for j in range(D // 16):
                    rows_v[0, i, pl.ds(j * 16, 16)] = ovec
                return carry

            lax.fori_loop(0, CHUNK, ones_fill, None)

        def zero_copy(i, carry):
            pltpu.sync_copy(zbuf, agg_sh.at[pl.ds(s * RPT + i * 16, 16)])
            return carry

        lax.fori_loop(0, RPT // 16, zero_copy, None)

        @pl.when(s == NS - 1)
        def _():
            pltpu.sync_copy(zbuf, agg_sh.at[pl.ds(N - 16, 16)])

        plsc.subcore_barrier()

        ebase = (c * NS + s) * EDGES_PER_TILE

        def idx_off(g):
            return ebase + g * CHUNK

        if not ones_mode:
            pltpu.sync_copy(src_hbm.at[pl.ds(idx_off(0), CHUNK)], src_v.at[0])
        pltpu.sync_copy(dst_hbm.at[pl.ds(idx_off(0), CHUNK)], dst_v.at[0])
        if not ones_mode:
            pltpu.async_copy(table_hbm.at[src_v.at[0]], rows_v.at[0], gsem)
            pltpu.async_copy(src_hbm.at[pl.ds(idx_off(1), CHUNK)],
                             src_v.at[1], isem)
        pltpu.async_copy(dst_hbm.at[pl.ds(idx_off(1), CHUNK)],
                         dst_v.at[1], isem)

        def body(g, carry):
            b = jnp.bitwise_and(g, 1)
            nb = 1 - b

            @pl.when(g < NCHUNKS - 1)
            def _():
                if not ones_mode:
                    pltpu.make_async_copy(
                        src_hbm.at[pl.ds(idx_off(g + 1), CHUNK)],
                        src_v.at[nb], isem).wait()
                pltpu.make_async_copy(
                    dst_hbm.at[pl.ds(idx_off(g + 1), CHUNK)],
                    dst_v.at[nb], isem).wait()
                if not ones_mode:
                    pltpu.async_copy(table_hbm.at[src_v.at[nb]],
                                     rows_v.at[nb], gsem)

            if not ones_mode:
                pltpu.make_async_copy(table_hbm.at[src_v.at[b]],
                                      rows_v.at[b], gsem).wait()
                pltpu.sync_copy(rows_v.at[b], agg_sh.at[dst_v.at[b]],
                                add=True)
            else:
                pltpu.sync_copy(rows_v.at[0], agg_sh.at[dst_v.at[b]],
                                add=True)

            @pl.when(g < NCHUNKS - 2)
            def _():
                if not ones_mode:
                    pltpu.async_copy(src_hbm.at[pl.ds(idx_off(g + 2), CHUNK)],
                                     src_v.at[b], isem)
                pltpu.async_copy(dst_hbm.at[pl.ds(idx_off(g + 2), CHUNK)],
                                 dst_v.at[b], isem)

            return carry

        lax.fori_loop(0, NCHUNKS, body, None)
        plsc.subcore_barrier()

        pltpu.sync_copy(
            agg_sh.at[pl.ds(s * RPT, RPT)],
            out_hbm.at[c].at[pl.ds(s * RPT, RPT)],
        )

        @pl.when(s == NS - 1)
        def _():
            pltpu.sync_copy(
                agg_sh.at[pl.ds(N - 16, 16)],
                out_hbm.at[c].at[pl.ds(N - 16, 16)],
            )

    return seg_sum


_seg_sum_feat = _build_seg_sum(False)
_seg_sum_deg = _build_seg_sum(True)


def _dense(p, h, d0, d1, Wl, bl, Wr, relu):
    BLK = 1000

    def body(p_ref, h_ref, d0_ref, d1_ref, wl_ref, bl_ref, wr_ref, o_ref):
        deg = jnp.maximum(d0_ref[...] + d1_ref[...], 1.0)
        agg = (p_ref[0] + p_ref[1]) / deg
        acc = lax.dot_general(
            agg, wl_ref[...], (((1,), (1,)), ((), ())),
            preferred_element_type=jnp.float32,
        )
        acc = acc + bl_ref[...] + lax.dot_general(
            h_ref[...], wr_ref[...], (((1,), (1,)), ((), ())),
            preferred_element_type=jnp.float32,
        )
        if relu:
            acc = jnp.maximum(acc, 0.0)
        o_ref[...] = acc

    return pl.pallas_call(
        body,
        grid=(N // BLK,),
        in_specs=[
            pl.BlockSpec((NC, BLK, D), lambda i: (0, i, 0)),
            pl.BlockSpec((BLK, D), lambda i: (i, 0)),
            pl.BlockSpec((BLK, 1), lambda i: (i, 0)),
            pl.BlockSpec((BLK, 1), lambda i: (i, 0)),
            pl.BlockSpec((D, D), lambda i: (0, 0)),
            pl.BlockSpec((1, D), lambda i: (0, 0)),
            pl.BlockSpec((D, D), lambda i: (0, 0)),
        ],
        out_specs=pl.BlockSpec((BLK, D), lambda i: (i, 0)),
        out_shape=jax.ShapeDtypeStruct((N, D), jnp.float32),
    )(p, h, d0, d1, Wl, bl, Wr)


def kernel(x, edge_index, Wl1, bl1, Wr1, Wl2, bl2, Wr2, Wl3, bl3, Wr3,
           Wl4, bl4, Wr4, Wl5, bl5, Wr5):
    src = edge_index[0]
    dst = edge_index[1]

    degp = _seg_sum_deg(dst)
    d0 = degp[0, :, 0:1]
    d1 = degp[1, :, 0:1]

    layers = [
        (Wl1, bl1, Wr1, True),
        (Wl2, bl2, Wr2, True),
        (Wl3, bl3, Wr3, True),
        (Wl4, bl4, Wr4, True),
        (Wl5, bl5, Wr5, False),
    ]
    h = x
    for Wl, bl, Wr, relu in layers:
        p = _seg_sum_feat(h, src, dst)
        h = _dense(p, h, d0, d1, Wl, bl.reshape(1, D), Wr, relu)
    return h


# speedup vs baseline: 9.2343x; 1.1351x over previous
import functools

import jax
import jax.numpy as jnp
from jax import lax
from jax.experimental import pallas as pl
from jax.experimental.pallas import tpu as pltpu
from jax.experimental.pallas import tpu_sc as plsc

N = 10000
E = 320000
D = 128

NC = 2
NS = 16
CHUNK = 80
EDGES_PER_TILE = E // (NC * NS)
NCHUNKS = EDGES_PER_TILE // CHUNK
RPT = 624


def _build_seg_sum(ones_mode: bool):
    mesh = plsc.VectorSubcoreMesh(
        core_axis_name="c", subcore_axis_name="s", num_cores=NC, num_subcores=NS
    )

    @functools.partial(
        pl.kernel,
        out_type=jax.ShapeDtypeStruct((NC, N, D), jnp.float32),
        mesh=mesh,
        scratch_types=[
            pltpu.VMEM((2, CHUNK), jnp.int32),
            pltpu.VMEM((4, CHUNK), jnp.int32),
            pltpu.VMEM((2, CHUNK, D), jnp.float32),
            pltpu.VMEM((16, D), jnp.float32),
            pltpu.SemaphoreType.DMA,
            pltpu.SemaphoreType.DMA,
            pltpu.SemaphoreType.DMA,
            pltpu.VMEM_SHARED((N, D), jnp.float32),
        ],
    )
    def seg_sum(*args):
        if ones_mode:
            (dst_hbm, out_hbm, src_v, dst_v, rows_v, zbuf, gsem, isem,
             ssem, agg_sh) = args
        else:
            (table_hbm, src_hbm, dst_hbm, out_hbm, src_v, dst_v, rows_v,
             zbuf, gsem, isem, ssem, agg_sh) = args
        c = lax.axis_index("c")
        s = lax.axis_index("s")

        zvec = jnp.zeros((16,), jnp.float32)
        ovec = jnp.ones((16,), jnp.float32)

        def zero_fill(i, carry):
            for j in range(D // 16):
                zbuf[i, pl.ds(j * 16, 16)] = zvec
            return carry

        lax.fori_loop(0, 16, zero_fill, None)

        if ones_mode:
            def ones_fill(i, carry):
                for j in range(D // 16):
                    rows_v[0, i, pl.ds(j * 16, 16)] = ovec
                return carry

            lax.fori_loop(0, CHUNK, ones_fill, None)

        def zero_copy(i, carry):
            pltpu.sync_copy(zbuf, agg_sh.at[pl.ds(s * RPT + i * 16, 16)])
            return carry

        lax.fori_loop(0, RPT // 16, zero_copy, None)

        @pl.when(s == NS - 1)
        def _():
            pltpu.sync_copy(zbuf, agg_sh.at[pl.ds(N - 16, 16)])

        plsc.subcore_barrier()

        ebase = (c * NS + s) * EDGES_PER_TILE

        def idx_off(g):
            return ebase + g * CHUNK

        if not ones_mode:
            pltpu.sync_copy(src_hbm.at[pl.ds(idx_off(0), CHUNK)], src_v.at[0])
        pltpu.sync_copy(dst_hbm.at[pl.ds(idx_off(0), CHUNK)], dst_v.at[0])
        if not ones_mode:
            pltpu.async_copy(table_hbm.at[src_v.at[0]], rows_v.at[0], gsem)
            pltpu.async_copy(src_hbm.at[pl.ds(idx_off(1), CHUNK)],
                             src_v.at[1], isem)
        pltpu.async_copy(dst_hbm.at[pl.ds(idx_off(1), CHUNK)],
                         dst_v.at[1], isem)

        def body(g, carry):
            b = jnp.bitwise_and(g, 1)
            nb = 1 - b
            d4 = jnp.bitwise_and(g, 3)
            d4p = jnp.bitwise_and(g + 2, 3)
            d4m = jnp.bitwise_and(g - 1, 3)

            @pl.when(g < NCHUNKS - 1)
            def _():
                if not ones_mode:
                    pltpu.make_async_copy(
                        src_hbm.at[pl.ds(idx_off(g + 1), CHUNK)],
                        src_v.at[nb], isem).wait()
                pltpu.make_async_copy(
                    dst_hbm.at[pl.ds(idx_off(g + 1), CHUNK)],
                    dst_v.at[nb], isem).wait()
                if not ones_mode:
                    @pl.when(g >= 1)
                    def _():
                        pltpu.make_async_copy(
                            rows_v.at[nb], agg_sh.at[dst_v.at[d4m]],
                            ssem).wait()

                    pltpu.async_copy(table_hbm.at[src_v.at[nb]],
                                     rows_v.at[nb], gsem)

            if not ones_mode:
                pltpu.make_async_copy(table_hbm.at[src_v.at[b]],
                                      rows_v.at[b], gsem).wait()
                pltpu.async_copy(rows_v.at[b], agg_sh.at[dst_v.at[d4]],
                                 ssem, add=True)
            else:
                pltpu.sync_copy(rows_v.at[0], agg_sh.at[dst_v.at[d4]],
                                add=True)

            @pl.when(g < NCHUNKS - 2)
            def _():
                if not ones_mode:
                    pltpu.async_copy(src_hbm.at[pl.ds(idx_off(g + 2), CHUNK)],
                                     src_v.at[b], isem)
                pltpu.async_copy(dst_hbm.at[pl.ds(idx_off(g + 2), CHUNK)],
                                 dst_v.at[d4p], isem)

            return carry

        lax.fori_loop(0, NCHUNKS, body, None)

        if not ones_mode:
            pltpu.make_async_copy(
                rows_v.at[(NCHUNKS - 2) % 2],
                agg_sh.at[dst_v.at[(NCHUNKS - 2) % 4]], ssem).wait()
            pltpu.make_async_copy(
                rows_v.at[(NCHUNKS - 1) % 2],
                agg_sh.at[dst_v.at[(NCHUNKS - 1) % 4]], ssem).wait()
        plsc.subcore_barrier()

        pltpu.sync_copy(
            agg_sh.at[pl.ds(s * RPT, RPT)],
            out_hbm.at[c].at[pl.ds(s * RPT, RPT)],
        )

        @pl.when(s == NS - 1)
        def _():
            pltpu.sync_copy(
                agg_sh.at[pl.ds(N - 16, 16)],
                out_hbm.at[c].at[pl.ds(N - 16, 16)],
            )

    return seg_sum


_seg_sum_feat = _build_seg_sum(False)
_seg_sum_deg = _build_seg_sum(True)


def _dense(p, h, d0, d1, Wl, bl, Wr, relu):
    BLK = 1000

    def body(p_ref, h_ref, d0_ref, d1_ref, wl_ref, bl_ref, wr_ref, o_ref):
        deg = jnp.maximum(d0_ref[...] + d1_ref[...], 1.0)
        agg = (p_ref[0] + p_ref[1]) / deg
        acc = lax.dot_general(
            agg, wl_ref[...], (((1,), (1,)), ((), ())),
            preferred_element_type=jnp.float32,
        )
        acc = acc + bl_ref[...] + lax.dot_general(
            h_ref[...], wr_ref[...], (((1,), (1,)), ((), ())),
            preferred_element_type=jnp.float32,
        )
        if relu:
            acc = jnp.maximum(acc, 0.0)
        o_ref[...] = acc

    return pl.pallas_call(
        body,
        grid=(N // BLK,),
        in_specs=[
            pl.BlockSpec((NC, BLK, D), lambda i: (0, i, 0)),
            pl.BlockSpec((BLK, D), lambda i: (i, 0)),
            pl.BlockSpec((BLK, 1), lambda i: (i, 0)),
            pl.BlockSpec((BLK, 1), lambda i: (i, 0)),
            pl.BlockSpec((D, D), lambda i: (0, 0)),
            pl.BlockSpec((1, D), lambda i: (0, 0)),
            pl.BlockSpec((D, D), lambda i: (0, 0)),
        ],
        out_specs=pl.BlockSpec((BLK, D), lambda i: (i, 0)),
        out_shape=jax.ShapeDtypeStruct((N, D), jnp.float32),
    )(p, h, d0, d1, Wl, bl, Wr)


def kernel(x, edge_index, Wl1, bl1, Wr1, Wl2, bl2, Wr2, Wl3, bl3, Wr3,
           Wl4, bl4, Wr4, Wl5, bl5, Wr5):
    src = edge_index[0]
    dst = edge_index[1]

    degp = _seg_sum_deg(dst)
    d0 = degp[0, :, 0:1]
    d1 = degp[1, :, 0:1]

    layers = [
        (Wl1, bl1, Wr1, True),
        (Wl2, bl2, Wr2, True),
        (Wl3, bl3, Wr3, True),
        (Wl4, bl4, Wr4, True),
        (Wl5, bl5, Wr5, False),
    ]
    h = x
    for Wl, bl, Wr, relu in layers:
        p = _seg_sum_feat(h, src, dst)
        h = _dense(p, h, d0, d1, Wl, bl.reshape(1, D), Wr, relu)
    return h


# speedup vs baseline: 9.4818x; 1.0268x over previous
import functools

import jax
import jax.numpy as jnp
from jax import lax
from jax.experimental import pallas as pl
from jax.experimental.pallas import tpu as pltpu
from jax.experimental.pallas import tpu_sc as plsc

N = 10000
E = 320000
D = 128

NC = 2
NS = 16
CHUNK = 80
EDGES_PER_TILE = E // (NC * NS)
NCHUNKS = EDGES_PER_TILE // CHUNK
RPT = 624


def _build_seg_sum(ones_mode: bool):
    mesh = plsc.VectorSubcoreMesh(
        core_axis_name="c", subcore_axis_name="s", num_cores=NC, num_subcores=NS
    )

    @functools.partial(
        pl.kernel,
        out_type=jax.ShapeDtypeStruct((NC, N, D), jnp.float32),
        mesh=mesh,
        scratch_types=[
            pltpu.VMEM((2, CHUNK), jnp.int32),
            pltpu.VMEM((4, CHUNK), jnp.int32),
            pltpu.VMEM((2, CHUNK, D), jnp.float32),
            pltpu.VMEM((48, D), jnp.float32),
            pltpu.SemaphoreType.DMA,
            pltpu.SemaphoreType.DMA,
            pltpu.SemaphoreType.DMA,
            pltpu.SemaphoreType.DMA,
            pltpu.VMEM_SHARED((N, D), jnp.float32),
        ],
    )
    def seg_sum(*args):
        if ones_mode:
            (dst_hbm, out_hbm, src_v, dst_v, rows_v, zbuf, gsem, isem,
             ssem, zsem, agg_sh) = args
        else:
            (table_hbm, src_hbm, dst_hbm, out_hbm, src_v, dst_v, rows_v,
             zbuf, gsem, isem, ssem, zsem, agg_sh) = args
        c = lax.axis_index("c")
        s = lax.axis_index("s")

        zvec = jnp.zeros((16,), jnp.float32)
        ovec = jnp.ones((16,), jnp.float32)

        def zero_fill(i, carry):
            for j in range(D // 16):
                zbuf[i, pl.ds(j * 16, 16)] = zvec
            return carry

        lax.fori_loop(0, 48, zero_fill, None)

        for q in range(RPT // 48):
            pltpu.async_copy(
                zbuf, agg_sh.at[pl.ds(s * RPT + q * 48, 48)], zsem)

        @pl.when(s == NS - 1)
        def _():
            pltpu.async_copy(zbuf.at[pl.ds(0, 16)],
                             agg_sh.at[pl.ds(N - 16, 16)], zsem)

        if ones_mode:
            def ones_fill(i, carry):
                for j in range(D // 16):
                    rows_v[0, i, pl.ds(j * 16, 16)] = ovec
                return carry

            lax.fori_loop(0, CHUNK, ones_fill, None)


        ebase = (c * NS + s) * EDGES_PER_TILE

        def idx_off(g):
            return ebase + g * CHUNK

        if not ones_mode:
            pltpu.sync_copy(src_hbm.at[pl.ds(idx_off(0), CHUNK)], src_v.at[0])
        pltpu.sync_copy(dst_hbm.at[pl.ds(idx_off(0), CHUNK)], dst_v.at[0])
        if not ones_mode:
            pltpu.async_copy(table_hbm.at[src_v.at[0]], rows_v.at[0], gsem)
            pltpu.async_copy(src_hbm.at[pl.ds(idx_off(1), CHUNK)],
                             src_v.at[1], isem)
        pltpu.async_copy(dst_hbm.at[pl.ds(idx_off(1), CHUNK)],
                         dst_v.at[1], isem)

        for q in range(RPT // 48):
            pltpu.make_async_copy(
                zbuf, agg_sh.at[pl.ds(s * RPT + q * 48, 48)], zsem).wait()

        @pl.when(s == NS - 1)
        def _():
            pltpu.make_async_copy(zbuf.at[pl.ds(0, 16)],
                                  agg_sh.at[pl.ds(N - 16, 16)], zsem).wait()

        plsc.subcore_barrier()

        def body(g, carry):
            b = jnp.bitwise_and(g, 1)
            nb = 1 - b
            d4 = jnp.bitwise_and(g, 3)
            d4p = jnp.bitwise_and(g + 2, 3)
            d4m = jnp.bitwise_and(g - 1, 3)

            @pl.when(g < NCHUNKS - 1)
            def _():
                if not ones_mode:
                    pltpu.make_async_copy(
                        src_hbm.at[pl.ds(idx_off(g + 1), CHUNK)],
                        src_v.at[nb], isem).wait()
                pltpu.make_async_copy(
                    dst_hbm.at[pl.ds(idx_off(g + 1), CHUNK)],
                    dst_v.at[nb], isem).wait()
                if not ones_mode:
                    @pl.when(g >= 1)
                    def _():
                        pltpu.make_async_copy(
                            rows_v.at[nb], agg_sh.at[dst_v.at[d4m]],
                            ssem).wait()

                    pltpu.async_copy(table_hbm.at[src_v.at[nb]],
                                     rows_v.at[nb], gsem)

            if not ones_mode:
                pltpu.make_async_copy(table_hbm.at[src_v.at[b]],
                                      rows_v.at[b], gsem).wait()
                pltpu.async_copy(rows_v.at[b], agg_sh.at[dst_v.at[d4]],
                                 ssem, add=True)
            else:
                pltpu.sync_copy(rows_v.at[0], agg_sh.at[dst_v.at[d4]],
                                add=True)

            @pl.when(g < NCHUNKS - 2)
            def _():
                if not ones_mode:
                    pltpu.async_copy(src_hbm.at[pl.ds(idx_off(g + 2), CHUNK)],
                                     src_v.at[b], isem)
                pltpu.async_copy(dst_hbm.at[pl.ds(idx_off(g + 2), CHUNK)],
                                 dst_v.at[d4p], isem)

            return carry

        lax.fori_loop(0, NCHUNKS, body, None)

        if not ones_mode:
            pltpu.make_async_copy(
                rows_v.at[(NCHUNKS - 2) % 2],
                agg_sh.at[dst_v.at[(NCHUNKS - 2) % 4]], ssem).wait()
            pltpu.make_async_copy(
                rows_v.at[(NCHUNKS - 1) % 2],
                agg_sh.at[dst_v.at[(NCHUNKS - 1) % 4]], ssem).wait()
        plsc.subcore_barrier()

        pltpu.sync_copy(
            agg_sh.at[pl.ds(s * RPT, RPT)],
            out_hbm.at[c].at[pl.ds(s * RPT, RPT)],
        )

        @pl.when(s == NS - 1)
        def _():
            pltpu.sync_copy(
                agg_sh.at[pl.ds(N - 16, 16)],
                out_hbm.at[c].at[pl.ds(N - 16, 16)],
            )

    return seg_sum


_seg_sum_feat = _build_seg_sum(False)

_seg_sum_deg = _build_seg_sum(True)


def _dense(p, h, d0, d1, Wl, bl, Wr, relu):
    BLK = 1000

    def body(p_ref, h_ref, d0_ref, d1_ref, wl_ref, bl_ref, wr_ref, o_ref):
        deg = jnp.maximum(d0_ref[...] + d1_ref[...], 1.0)
        agg = (p_ref[0] + p_ref[1]) / deg
        acc = lax.dot_general(
            agg, wl_ref[...], (((1,), (1,)), ((), ())),
            preferred_element_type=jnp.float32,
        )
        acc = acc + bl_ref[...] + lax.dot_general(
            h_ref[...], wr_ref[...], (((1,), (1,)), ((), ())),
            preferred_element_type=jnp.float32,
        )
        if relu:
            acc = jnp.maximum(acc, 0.0)
        o_ref[...] = acc

    return pl.pallas_call(
        body,
        grid=(N // BLK,),
        in_specs=[
            pl.BlockSpec((NC, BLK, D), lambda i: (0, i, 0)),
            pl.BlockSpec((BLK, D), lambda i: (i, 0)),
            pl.BlockSpec((BLK, 1), lambda i: (i, 0)),
            pl.BlockSpec((BLK, 1), lambda i: (i, 0)),
            pl.BlockSpec((D, D), lambda i: (0, 0)),
            pl.BlockSpec((1, D), lambda i: (0, 0)),
            pl.BlockSpec((D, D), lambda i: (0, 0)),
        ],
        out_specs=pl.BlockSpec((BLK, D), lambda i: (i, 0)),
        out_shape=jax.ShapeDtypeStruct((N, D), jnp.float32),
    )(p, h, d0, d1, Wl, bl, Wr)


def kernel(x, edge_index, Wl1, bl1, Wr1, Wl2, bl2, Wr2, Wl3, bl3, Wr3,
           Wl4, bl4, Wr4, Wl5, bl5, Wr5):
    src = edge_index[0]
    dst = edge_index[1]

    degp = _seg_sum_deg(dst)
    d0 = degp[0, :, 0:1]
    d1 = degp[1, :, 0:1]

    layers = [
        (Wl1, bl1, Wr1, True),
        (Wl2, bl2, Wr2, True),
        (Wl3, bl3, Wr3, True),
        (Wl4, bl4, Wr4, True),
        (Wl5, bl5, Wr5, False),
    ]
    h = x
    for Wl, bl, Wr, relu in layers:
        p = _seg_sum_feat(h, src, dst)
        h = _dense(p, h, d0, d1, Wl, bl.reshape(1, D), Wr, relu)
    return h


# speedup vs baseline: 10.6265x; 1.1207x over previous
import functools

import jax
import jax.numpy as jnp
from jax import lax
from jax.experimental import pallas as pl
from jax.experimental.pallas import tpu as pltpu
from jax.experimental.pallas import tpu_sc as plsc

N = 10000
E = 320000
D = 128

NC = 2
NS = 16
CHUNK = 128
NCHUNKS = 78
EDGES_PER_TILE = NCHUNKS * CHUNK
EXTRA_BASE = NC * NS * EDGES_PER_TILE
NEXTRA = (E - EXTRA_BASE) // CHUNK
RPT = 624


def _build_seg_sum(ones_mode: bool):
    mesh = plsc.VectorSubcoreMesh(
        core_axis_name="c", subcore_axis_name="s", num_cores=NC, num_subcores=NS
    )

    @functools.partial(
        pl.kernel,
        out_type=jax.ShapeDtypeStruct((NC, N, D), jnp.float32),
        mesh=mesh,
        scratch_types=[
            pltpu.VMEM((2, CHUNK), jnp.int32),
            pltpu.VMEM((4, CHUNK), jnp.int32),
            pltpu.VMEM((2, CHUNK, D), jnp.float32),
            pltpu.VMEM((48, D), jnp.float32),
            pltpu.SemaphoreType.DMA,
            pltpu.SemaphoreType.DMA,
            pltpu.SemaphoreType.DMA,
            pltpu.SemaphoreType.DMA,
            pltpu.VMEM_SHARED((N, D), jnp.float32),
        ],
    )
    def seg_sum(*args):
        if ones_mode:
            (dst_hbm, out_hbm, src_v, dst_v, rows_v, zbuf, gsem, isem,
             ssem, zsem, agg_sh) = args
        else:
            (table_hbm, src_hbm, dst_hbm, out_hbm, src_v, dst_v, rows_v,
             zbuf, gsem, isem, ssem, zsem, agg_sh) = args
        c = lax.axis_index("c")
        s = lax.axis_index("s")

        zvec = jnp.zeros((16,), jnp.float32)
        ovec = jnp.ones((16,), jnp.float32)

        def zero_fill(i, carry):
            for j in range(D // 16):
                zbuf[i, pl.ds(j * 16, 16)] = zvec
            return carry

        lax.fori_loop(0, 48, zero_fill, None)

        for q in range(RPT // 48):
            pltpu.async_copy(
                zbuf, agg_sh.at[pl.ds(s * RPT + q * 48, 48)], zsem)

        @pl.when(s == NS - 1)
        def _():
            pltpu.async_copy(zbuf.at[pl.ds(0, 16)],
                             agg_sh.at[pl.ds(N - 16, 16)], zsem)

        if ones_mode:
            def ones_fill(i, carry):
                for j in range(D // 16):
                    rows_v[0, i, pl.ds(j * 16, 16)] = ovec
                return carry

            lax.fori_loop(0, CHUNK, ones_fill, None)


        ebase = (c * NS + s) * EDGES_PER_TILE

        def idx_off(g):
            return ebase + g * CHUNK

        if not ones_mode:
            pltpu.sync_copy(src_hbm.at[pl.ds(idx_off(0), CHUNK)], src_v.at[0])
        pltpu.sync_copy(dst_hbm.at[pl.ds(idx_off(0), CHUNK)], dst_v.at[0])
        if not ones_mode:
            pltpu.async_copy(table_hbm.at[src_v.at[0]], rows_v.at[0], gsem)
            pltpu.async_copy(src_hbm.at[pl.ds(idx_off(1), CHUNK)],
                             src_v.at[1], isem)
        pltpu.async_copy(dst_hbm.at[pl.ds(idx_off(1), CHUNK)],
                         dst_v.at[1], isem)

        for q in range(RPT // 48):
            pltpu.make_async_copy(
                zbuf, agg_sh.at[pl.ds(s * RPT + q * 48, 48)], zsem).wait()

        @pl.when(s == NS - 1)
        def _():
            pltpu.make_async_copy(zbuf.at[pl.ds(0, 16)],
                                  agg_sh.at[pl.ds(N - 16, 16)], zsem).wait()

        plsc.subcore_barrier()

        def body(g, carry):
            b = jnp.bitwise_and(g, 1)
            nb = 1 - b
            d4 = jnp.bitwise_and(g, 3)
            d4p = jnp.bitwise_and(g + 2, 3)
            d4m = jnp.bitwise_and(g - 1, 3)

            @pl.when(g < NCHUNKS - 1)
            def _():
                if not ones_mode:
                    pltpu.make_async_copy(
                        src_hbm.at[pl.ds(idx_off(g + 1), CHUNK)],
                        src_v.at[nb], isem).wait()
                pltpu.make_async_copy(
                    dst_hbm.at[pl.ds(idx_off(g + 1), CHUNK)],
                    dst_v.at[nb], isem).wait()
                if not ones_mode:
                    @pl.when(g >= 1)
                    def _():
                        pltpu.make_async_copy(
                            rows_v.at[nb], agg_sh.at[dst_v.at[d4m]],
                            ssem).wait()

                    pltpu.async_copy(table_hbm.at[src_v.at[nb]],
                                     rows_v.at[nb], gsem)

            if not ones_mode:
                pltpu.make_async_copy(table_hbm.at[src_v.at[b]],
                                      rows_v.at[b], gsem).wait()
                pltpu.async_copy(rows_v.at[b], agg_sh.at[dst_v.at[d4]],
                                 ssem, add=True)
            else:
                pltpu.sync_copy(rows_v.at[0], agg_sh.at[dst_v.at[d4]],
                                add=True)

            @pl.when(g < NCHUNKS - 2)
            def _():
                if not ones_mode:
                    pltpu.async_copy(src_hbm.at[pl.ds(idx_off(g + 2), CHUNK)],
                                     src_v.at[b], isem)
                pltpu.async_copy(dst_hbm.at[pl.ds(idx_off(g + 2), CHUNK)],
                                 dst_v.at[d4p], isem)

            return carry

        lax.fori_loop(0, NCHUNKS, body, None)

        if not ones_mode:
            pltpu.make_async_copy(
                rows_v.at[(NCHUNKS - 2) % 2],
                agg_sh.at[dst_v.at[(NCHUNKS - 2) % 4]], ssem).wait()
            pltpu.make_async_copy(
                rows_v.at[(NCHUNKS - 1) % 2],
                agg_sh.at[dst_v.at[(NCHUNKS - 1) % 4]], ssem).wait()

        tid = c * NS + s

        @pl.when(tid < NEXTRA)
        def _():
            off = EXTRA_BASE + tid * CHUNK
            if not ones_mode:
                pltpu.sync_copy(src_hbm.at[pl.ds(off, CHUNK)], src_v.at[0])
            pltpu.sync_copy(dst_hbm.at[pl.ds(off, CHUNK)], dst_v.at[0])
            if not ones_mode:
                pltpu.async_copy(table_hbm.at[src_v.at[0]],
                                 rows_v.at[0], gsem).wait()
            pltpu.sync_copy(rows_v.at[0], agg_sh.at[dst_v.at[0]], add=True)

        plsc.subcore_barrier()

        pltpu.sync_copy(
            agg_sh.at[pl.ds(s * RPT, RPT)],
            out_hbm.at[c].at[pl.ds(s * RPT, RPT)],
        )

        @pl.when(s == NS - 1)
        def _():
            pltpu.sync_copy(
                agg_sh.at[pl.ds(N - 16, 16)],
                out_hbm.at[c].at[pl.ds(N - 16, 16)],
            )

    return seg_sum


_seg_sum_feat = _build_seg_sum(False)

_seg_sum_deg = _build_seg_sum(True)


def _dense(p, h, d0, d1, Wl, bl, Wr, relu):
    BLK = 1000

    def body(p_ref, h_ref, d0_ref, d1_ref, wl_ref, bl_ref, wr_ref, o_ref):
        deg = jnp.maximum(d0_ref[...] + d1_ref[...], 1.0)
        agg = (p_ref[0] + p_ref[1]) / deg
        acc = lax.dot_general(
            agg, wl_ref[...], (((1,), (1,)), ((), ())),
            preferred_element_type=jnp.float32,
        )
        acc = acc + bl_ref[...] + lax.dot_general(
            h_ref[...], wr_ref[...], (((1,), (1,)), ((), ())),
            preferred_element_type=jnp.float32,
        )
        if relu:
            acc = jnp.maximum(acc, 0.0)
        o_ref[...] = acc

    return pl.pallas_call(
        body,
        grid=(N // BLK,),
        in_specs=[
            pl.BlockSpec((NC, BLK, D), lambda i: (0, i, 0)),
            pl.BlockSpec((BLK, D), lambda i: (i, 0)),
            pl.BlockSpec((BLK, 1), lambda i: (i, 0)),
            pl.BlockSpec((BLK, 1), lambda i: (i, 0)),
            pl.BlockSpec((D, D), lambda i: (0, 0)),
            pl.BlockSpec((1, D), lambda i: (0, 0)),
            pl.BlockSpec((D, D), lambda i: (0, 0)),
        ],
        out_specs=pl.BlockSpec((BLK, D), lambda i: (i, 0)),
        out_shape=jax.ShapeDtypeStruct((N, D), jnp.float32),
    )(p, h, d0, d1, Wl, bl, Wr)


def kernel(x, edge_index, Wl1, bl1, Wr1, Wl2, bl2, Wr2, Wl3, bl3, Wr3,
           Wl4, bl4, Wr4, Wl5, bl5, Wr5):
    src = edge_index[0]
    dst = edge_index[1]

    degp = _seg_sum_deg(dst)
    d0 = degp[0, :, 0:1]
    d1 = degp[1, :, 0:1]

    layers = [
        (Wl1, bl1, Wr1, True),
        (Wl2, bl2, Wr2, True),
        (Wl3, bl3, Wr3, True),
        (Wl4, bl4, Wr4, True),
        (Wl5, bl5, Wr5, False),
    ]
    h = x
    for Wl, bl, Wr, relu in layers:
        p = _seg_sum_feat(h, src, dst)
        h = _dense(p, h, d0, d1, Wl, bl.reshape(1, D), Wr, relu)
    return h


# speedup vs baseline: 11.2808x; 1.0616x over previous
import functools

import jax
import jax.numpy as jnp
from jax import lax
from jax.experimental import pallas as pl
from jax.experimental.pallas import tpu as pltpu
from jax.experimental.pallas import tpu_sc as plsc

N = 10000
E = 320000
D = 128

NC = 2
NS = 16
CHUNK = 128
NCHUNKS = 78
EDGES_PER_TILE = NCHUNKS * CHUNK
EXTRA_BASE = NC * NS * EDGES_PER_TILE
NEXTRA = (E - EXTRA_BASE) // CHUNK
RPT = 624


def _build_seg_sum(ones_mode: bool, d: int = D):
    mesh = plsc.VectorSubcoreMesh(
        core_axis_name="c", subcore_axis_name="s", num_cores=NC, num_subcores=NS
    )

    @functools.partial(
        pl.kernel,
        out_type=jax.ShapeDtypeStruct((NC, N, d), jnp.float32),
        mesh=mesh,
        scratch_types=[
            pltpu.VMEM((2, CHUNK), jnp.int32),
            pltpu.VMEM((4, CHUNK), jnp.int32),
            pltpu.VMEM((2, CHUNK, d), jnp.float32),
            pltpu.VMEM((48, d), jnp.float32),
            pltpu.SemaphoreType.DMA,
            pltpu.SemaphoreType.DMA,
            pltpu.SemaphoreType.DMA,
            pltpu.SemaphoreType.DMA,
            pltpu.VMEM_SHARED((N, d), jnp.float32),
        ],
    )
    def seg_sum(*args):
        if ones_mode:
            (dst_hbm, out_hbm, src_v, dst_v, rows_v, zbuf, gsem, isem,
             ssem, zsem, agg_sh) = args
        else:
            (table_hbm, src_hbm, dst_hbm, out_hbm, src_v, dst_v, rows_v,
             zbuf, gsem, isem, ssem, zsem, agg_sh) = args
        c = lax.axis_index("c")
        s = lax.axis_index("s")

        zvec = jnp.zeros((16,), jnp.float32)
        ovec = jnp.ones((16,), jnp.float32)

        def zero_fill(i, carry):
            for j in range(d // 16):
                zbuf[i, pl.ds(j * 16, 16)] = zvec
            return carry

        lax.fori_loop(0, 48, zero_fill, None)

        for q in range(RPT // 48):
            pltpu.async_copy(
                zbuf, agg_sh.at[pl.ds(s * RPT + q * 48, 48)], zsem)

        @pl.when(s == NS - 1)
        def _():
            pltpu.async_copy(zbuf.at[pl.ds(0, 16)],
                             agg_sh.at[pl.ds(N - 16, 16)], zsem)

        if ones_mode:
            def ones_fill(i, carry):
                for j in range(d // 16):
                    rows_v[0, i, pl.ds(j * 16, 16)] = ovec
                return carry

            lax.fori_loop(0, CHUNK, ones_fill, None)


        ebase = (c * NS + s) * EDGES_PER_TILE

        def idx_off(g):
            return ebase + g * CHUNK

        if not ones_mode:
            pltpu.sync_copy(src_hbm.at[pl.ds(idx_off(0), CHUNK)], src_v.at[0])
        pltpu.sync_copy(dst_hbm.at[pl.ds(idx_off(0), CHUNK)], dst_v.at[0])
        if not ones_mode:
            pltpu.async_copy(table_hbm.at[src_v.at[0]], rows_v.at[0], gsem)
            pltpu.async_copy(src_hbm.at[pl.ds(idx_off(1), CHUNK)],
                             src_v.at[1], isem)
        pltpu.async_copy(dst_hbm.at[pl.ds(idx_off(1), CHUNK)],
                         dst_v.at[1], isem)

        for q in range(RPT // 48):
            pltpu.make_async_copy(
                zbuf, agg_sh.at[pl.ds(s * RPT + q * 48, 48)], zsem).wait()

        @pl.when(s == NS - 1)
        def _():
            pltpu.make_async_copy(zbuf.at[pl.ds(0, 16)],
                                  agg_sh.at[pl.ds(N - 16, 16)], zsem).wait()

        plsc.subcore_barrier()

        def body(g, carry):
            b = jnp.bitwise_and(g, 1)
            nb = 1 - b
            d4 = jnp.bitwise_and(g, 3)
            d4p = jnp.bitwise_and(g + 2, 3)
            d4m = jnp.bitwise_and(g - 1, 3)

            @pl.when(g < NCHUNKS - 1)
            def _():
                if not ones_mode:
                    pltpu.make_async_copy(
                        src_hbm.at[pl.ds(idx_off(g + 1), CHUNK)],
                        src_v.at[nb], isem).wait()
                pltpu.make_async_copy(
                    dst_hbm.at[pl.ds(idx_off(g + 1), CHUNK)],
                    dst_v.at[nb], isem).wait()
                if not ones_mode:
                    @pl.when(g >= 1)
                    def _():
                        pltpu.make_async_copy(
                            rows_v.at[nb], agg_sh.at[dst_v.at[d4m]],
                            ssem).wait()

                    pltpu.async_copy(table_hbm.at[src_v.at[nb]],
                                     rows_v.at[nb], gsem)

            if not ones_mode:
                pltpu.make_async_copy(table_hbm.at[src_v.at[b]],
                                      rows_v.at[b], gsem).wait()
                pltpu.async_copy(rows_v.at[b], agg_sh.at[dst_v.at[d4]],
                                 ssem, add=True)
            else:
                pltpu.sync_copy(rows_v.at[0], agg_sh.at[dst_v.at[d4]],
                                add=True)

            @pl.when(g < NCHUNKS - 2)
            def _():
                if not ones_mode:
                    pltpu.async_copy(src_hbm.at[pl.ds(idx_off(g + 2), CHUNK)],
                                     src_v.at[b], isem)
                pltpu.async_copy(dst_hbm.at[pl.ds(idx_off(g + 2), CHUNK)],
                                 dst_v.at[d4p], isem)

            return carry

        lax.fori_loop(0, NCHUNKS, body, None)

        if not ones_mode:
            pltpu.make_async_copy(
                rows_v.at[(NCHUNKS - 2) % 2],
                agg_sh.at[dst_v.at[(NCHUNKS - 2) % 4]], ssem).wait()
            pltpu.make_async_copy(
                rows_v.at[(NCHUNKS - 1) % 2],
                agg_sh.at[dst_v.at[(NCHUNKS - 1) % 4]], ssem).wait()

        tid = c * NS + s

        @pl.when(tid < NEXTRA)
        def _():
            off = EXTRA_BASE + tid * CHUNK
            if not ones_mode:
                pltpu.sync_copy(src_hbm.at[pl.ds(off, CHUNK)], src_v.at[0])
            pltpu.sync_copy(dst_hbm.at[pl.ds(off, CHUNK)], dst_v.at[0])
            if not ones_mode:
                pltpu.async_copy(table_hbm.at[src_v.at[0]],
                                 rows_v.at[0], gsem).wait()
            pltpu.sync_copy(rows_v.at[0], agg_sh.at[dst_v.at[0]], add=True)

        plsc.subcore_barrier()

        pltpu.sync_copy(
            agg_sh.at[pl.ds(s * RPT, RPT)],
            out_hbm.at[c].at[pl.ds(s * RPT, RPT)],
        )

        @pl.when(s == NS - 1)
        def _():
            pltpu.sync_copy(
                agg_sh.at[pl.ds(N - 16, 16)],
                out_hbm.at[c].at[pl.ds(N - 16, 16)],
            )

    return seg_sum


_seg_sum_feat = _build_seg_sum(False)

_seg_sum_deg = _build_seg_sum(True, d=16)


def _dense(p, h, d0, d1, Wl, bl, Wr, relu):
    BLK = 1000

    def body(p_ref, h_ref, d0_ref, d1_ref, wl_ref, bl_ref, wr_ref, o_ref):
        deg = jnp.maximum(d0_ref[...] + d1_ref[...], 1.0)
        agg = (p_ref[0] + p_ref[1]) / deg
        acc = lax.dot_general(
            agg, wl_ref[...], (((1,), (1,)), ((), ())),
            preferred_element_type=jnp.float32,
        )
        acc = acc + bl_ref[...] + lax.dot_general(
            h_ref[...], wr_ref[...], (((1,), (1,)), ((), ())),
            preferred_element_type=jnp.float32,
        )
        if relu:
            acc = jnp.maximum(acc, 0.0)
        o_ref[...] = acc

    return pl.pallas_call(
        body,
        grid=(N // BLK,),
        in_specs=[
            pl.BlockSpec((NC, BLK, D), lambda i: (0, i, 0)),
            pl.BlockSpec((BLK, D), lambda i: (i, 0)),
            pl.BlockSpec((BLK, 1), lambda i: (i, 0)),
            pl.BlockSpec((BLK, 1), lambda i: (i, 0)),
            pl.BlockSpec((D, D), lambda i: (0, 0)),
            pl.BlockSpec((1, D), lambda i: (0, 0)),
            pl.BlockSpec((D, D), lambda i: (0, 0)),
        ],
        out_specs=pl.BlockSpec((BLK, D), lambda i: (i, 0)),
        out_shape=jax.ShapeDtypeStruct((N, D), jnp.float32),
    )(p, h, d0, d1, Wl, bl, Wr)


def kernel(x, edge_index, Wl1, bl1, Wr1, Wl2, bl2, Wr2, Wl3, bl3, Wr3,
           Wl4, bl4, Wr4, Wl5, bl5, Wr5):
    src = edge_index[0]
    dst = edge_index[1]

    degp = _seg_sum_deg(dst)
    d0 = degp[0, :, 0:1]
    d1 = degp[1, :, 0:1]

    layers = [
        (Wl1, bl1, Wr1, True),
        (Wl2, bl2, Wr2, True),
        (Wl3, bl3, Wr3, True),
        (Wl4, bl4, Wr4, True),
        (Wl5, bl5, Wr5, False),
    ]
    h = x
    for Wl, bl, Wr, relu in layers:
        p = _seg_sum_feat(h, src, dst)
        h = _dense(p, h, d0, d1, Wl, bl.reshape(1, D), Wr, relu)
    return h
